# baseline (speedup 1.0000x reference)
import jax
import jax.numpy as jnp
import numpy as np
from jax import lax
from jax.experimental import pallas as pl
from jax.experimental.pallas import tpu as pltpu
from jax.experimental.pallas import tpu_sc as plsc

R = 128
N = 32768
K = 65
L = 16
NV = N // L
NC = 2
NS = 16
NW = NC * NS
ROWS_PER_W = R // NW
PIVOT = np.float32(2.5)
SIGN = np.int32(-(2**31))
LOW31 = np.int32(0x7FFFFFFF)
INT_MIN = np.int32(-(2**31))
UNROLL = 16


def _ckey(v):
    b = lax.bitcast_convert_type(v, jnp.int32)
    return jnp.where(b >= 0, b, b ^ LOW31)


def _row_threshold(row_v, cand_v):
    @plsc.parallel_loop(0, NV, unroll=UNROLL,
                        carry=jnp.zeros((L,), jnp.int32))
    def offv(i, off):
        v = row_v[pl.ds(i * L, L)]
        m = v > PIVOT
        mi = m.astype(jnp.int32)
        pos = (plsc.cumsum(mi) - mi) + off
        plsc.store_scatter(cand_v, [pos], v, mask=m)
        return off + plsc.all_reduce_population_count(m)

    cnt = jnp.max(offv)
    padpos = lax.iota(jnp.int32, L) + cnt
    plsc.store_scatter(cand_v, [padpos],
                       jnp.full((L,), -jnp.inf, jnp.float32))

    @pl.when(cnt < K)
    def _():
        @plsc.parallel_loop(0, NV, unroll=UNROLL)
        def _copy(i):
            cand_v[pl.ds(i * L, L)] = row_v[pl.ds(i * L, L)]

    cnt = jnp.where(cnt < K, N, cnt)
    nv = (cnt + (L - 1)) // L

    @plsc.parallel_loop(0, nv, unroll=4)
    def _tokey(i):
        kv = _ckey(cand_v[pl.ds(i * L, L)])
        cand_v[pl.ds(i * L, L)] = lax.bitcast_convert_type(kv, jnp.float32)

    def bit_body(bi, p):
        bit = jnp.left_shift(jnp.int32(1), 31 - bi)
        cand_t = p | bit
        cs = cand_t ^ SIGN

        def cbody(i, acc):
            kv = lax.bitcast_convert_type(cand_v[pl.ds(i * L, L)], jnp.int32)
            return acc + jnp.where(kv >= cs,
                                   jnp.full((L,), 1, jnp.int32),
                                   jnp.full((L,), 0, jnp.int32))

        accv = lax.fori_loop(0, nv, cbody, jnp.zeros((L,), jnp.int32))
        return jnp.where(jnp.sum(accv) >= K, cand_t, p)

    p_u = lax.fori_loop(0, 32, bit_body, jnp.int32(0))
    vkey = p_u ^ SIGN
    bsplat = jnp.full((L,), vkey, jnp.int32)
    bsplat = jnp.where(bsplat >= 0, bsplat, bsplat ^ LOW31)
    return lax.bitcast_convert_type(bsplat, jnp.float32)


def _sc_body(x_hbm, out_hbm, row0_v, row1_v, cand_v,
             in_sem0, in_sem1, out_sem0, out_sem1):
    wid = lax.axis_index("s") * NC + lax.axis_index("c")
    r0 = wid * ROWS_PER_W
    bufs = [row0_v, row1_v]
    in_sems = [in_sem0, in_sem1]
    out_sems = [out_sem0, out_sem1]

    copies_in = [None] * ROWS_PER_W
    copies_out = [None] * ROWS_PER_W
    copies_in[0] = pltpu.async_copy(x_hbm.at[r0], bufs[0], in_sems[0])
    for j in range(ROWS_PER_W):
        b = j % 2
        row_v = bufs[b]
        copies_in[j].wait()
        if j + 1 < ROWS_PER_W:
            if j >= 1:
                copies_out[j - 1].wait()
            copies_in[j + 1] = pltpu.async_copy(
                x_hbm.at[r0 + j + 1], bufs[1 - b], in_sems[1 - b])

        thr = _row_threshold(row_v, cand_v)

        @plsc.parallel_loop(0, NV, unroll=UNROLL)
        def _mask(i):
            v = row_v[pl.ds(i * L, L)]
            row_v[pl.ds(i * L, L)] = jnp.where(v > thr, v, jnp.float32(0.0))

        copies_out[j] = pltpu.async_copy(row_v, out_hbm.at[r0 + j],
                                         out_sems[b])
    copies_out[ROWS_PER_W - 2].wait()
    copies_out[ROWS_PER_W - 1].wait()


@jax.jit
def _ksparse_sc(x):
    mesh = plsc.VectorSubcoreMesh(core_axis_name="c", subcore_axis_name="s")
    return pl.kernel(
        _sc_body,
        out_type=jax.ShapeDtypeStruct((R, N), jnp.float32),
        mesh=mesh,
        compiler_params=pltpu.CompilerParams(needs_layout_passes=False),
        scratch_types=[
            pltpu.VMEM((N,), jnp.float32),
            pltpu.VMEM((N,), jnp.float32),
            pltpu.VMEM((N + L,), jnp.float32),
            pltpu.SemaphoreType.DMA,
            pltpu.SemaphoreType.DMA,
            pltpu.SemaphoreType.DMA,
            pltpu.SemaphoreType.DMA,
        ],
    )(x)


def kernel(inputs):
    return _ksparse_sc(inputs)


# speedup vs baseline: 1.1072x; 1.1072x over previous
import jax
import jax.numpy as jnp
import numpy as np
from jax import lax
from jax.experimental import pallas as pl
from jax.experimental.pallas import tpu as pltpu
from jax.experimental.pallas import tpu_sc as plsc

R = 128
N = 32768
K = 65
L = 16
NV = N // L
NC = 2
NS = 16
NW = NC * NS
ROWS_PER_W = R // NW
PIVOT = np.float32(2.5)
SIGN = np.int32(-(2**31))
LOW31 = np.int32(0x7FFFFFFF)
INT_MIN = np.int32(-(2**31))
UNROLL = 4


def _ckey(v):
    b = lax.bitcast_convert_type(v, jnp.int32)
    return jnp.where(b >= 0, b, b ^ LOW31)


def _row_threshold(row_v, cand_v):
    @plsc.parallel_loop(0, NV, unroll=UNROLL,
                        carry=jnp.zeros((L,), jnp.int32))
    def offv(i, off):
        v = row_v[pl.ds(i * L, L)]
        m = v > PIVOT
        mi = m.astype(jnp.int32)
        pos = (plsc.cumsum(mi) - mi) + off
        plsc.store_scatter(cand_v, [pos], v, mask=m)
        return off + plsc.all_reduce_population_count(m)

    cnt = jnp.max(offv)
    padpos = lax.iota(jnp.int32, L) + cnt
    plsc.store_scatter(cand_v, [padpos],
                       jnp.full((L,), -jnp.inf, jnp.float32))

    @pl.when(cnt < K)
    def _():
        @plsc.parallel_loop(0, NV, unroll=UNROLL)
        def _copy(i):
            cand_v[pl.ds(i * L, L)] = row_v[pl.ds(i * L, L)]

    cnt = jnp.where(cnt < K, N, cnt)
    nv = (cnt + (L - 1)) // L

    @plsc.parallel_loop(0, nv, unroll=4)
    def _tokey(i):
        kv = _ckey(cand_v[pl.ds(i * L, L)])
        cand_v[pl.ds(i * L, L)] = lax.bitcast_convert_type(kv, jnp.float32)

    def bit_body(bi, p):
        bit = jnp.left_shift(jnp.int32(1), 31 - bi)
        cand_t = p | bit
        cs = cand_t ^ SIGN

        def cbody(i, acc):
            kv = lax.bitcast_convert_type(cand_v[pl.ds(i * L, L)], jnp.int32)
            return acc + jnp.where(kv >= cs,
                                   jnp.full((L,), 1, jnp.int32),
                                   jnp.full((L,), 0, jnp.int32))

        accv = lax.fori_loop(0, nv, cbody, jnp.zeros((L,), jnp.int32))
        return jnp.where(jnp.sum(accv) >= K, cand_t, p)

    p_u = lax.fori_loop(0, 32, bit_body, jnp.int32(0))
    vkey = p_u ^ SIGN
    bsplat = jnp.full((L,), vkey, jnp.int32)
    bsplat = jnp.where(bsplat >= 0, bsplat, bsplat ^ LOW31)
    return lax.bitcast_convert_type(bsplat, jnp.float32)


def _sc_body(x_hbm, out_hbm, row0_v, row1_v, cand_v,
             in_sem0, in_sem1, out_sem0, out_sem1):
    wid = lax.axis_index("s") * NC + lax.axis_index("c")
    r0 = wid * ROWS_PER_W
    bufs = [row0_v, row1_v]
    in_sems = [in_sem0, in_sem1]
    out_sems = [out_sem0, out_sem1]

    copies_in = [None] * ROWS_PER_W
    copies_out = [None] * ROWS_PER_W
    copies_in[0] = pltpu.async_copy(x_hbm.at[r0], bufs[0], in_sems[0])
    for j in range(ROWS_PER_W):
        b = j % 2
        row_v = bufs[b]
        copies_in[j].wait()
        if j + 1 < ROWS_PER_W:
            if j >= 1:
                copies_out[j - 1].wait()
            copies_in[j + 1] = pltpu.async_copy(
                x_hbm.at[r0 + j + 1], bufs[1 - b], in_sems[1 - b])

        thr = _row_threshold(row_v, cand_v)

        @plsc.parallel_loop(0, NV, unroll=UNROLL)
        def _mask(i):
            v = row_v[pl.ds(i * L, L)]
            row_v[pl.ds(i * L, L)] = jnp.where(v > thr, v, jnp.float32(0.0))

        copies_out[j] = pltpu.async_copy(row_v, out_hbm.at[r0 + j],
                                         out_sems[b])
    copies_out[ROWS_PER_W - 2].wait()
    copies_out[ROWS_PER_W - 1].wait()


@jax.jit
def _ksparse_sc(x):
    mesh = plsc.VectorSubcoreMesh(core_axis_name="c", subcore_axis_name="s")
    return pl.kernel(
        _sc_body,
        out_type=jax.ShapeDtypeStruct((R, N), jnp.float32),
        mesh=mesh,
        compiler_params=pltpu.CompilerParams(needs_layout_passes=False),
        scratch_types=[
            pltpu.VMEM((N,), jnp.float32),
            pltpu.VMEM((N,), jnp.float32),
            pltpu.VMEM((N + L,), jnp.float32),
            pltpu.SemaphoreType.DMA,
            pltpu.SemaphoreType.DMA,
            pltpu.SemaphoreType.DMA,
            pltpu.SemaphoreType.DMA,
        ],
    )(x)


def kernel(inputs):
    return _ksparse_sc(inputs)


# speedup vs baseline: 1.2278x; 1.1089x over previous
import jax
import jax.numpy as jnp
import numpy as np
from jax import lax
from jax.experimental import pallas as pl
from jax.experimental.pallas import tpu as pltpu
from jax.experimental.pallas import tpu_sc as plsc

R = 128
N = 32768
K = 65
L = 16
NV = N // L
NC = 2
NS = 16
NW = NC * NS
ROWS_PER_W = R // NW
PIVOT = np.float32(2.5)
SIGN = np.int32(-(2**31))
LOW31 = np.int32(0x7FFFFFFF)
INT_MIN = np.int32(-(2**31))
UNROLL = 8


def _ckey(v):
    b = lax.bitcast_convert_type(v, jnp.int32)
    return jnp.where(b >= 0, b, b ^ LOW31)


def _row_threshold(row_v, cand_v):
    @plsc.parallel_loop(0, NV, unroll=UNROLL,
                        carry=jnp.zeros((L,), jnp.int32))
    def offv(i, off):
        v = row_v[pl.ds(i * L, L)]
        m = v > PIVOT
        mi = m.astype(jnp.int32)
        pos = (plsc.cumsum(mi) - mi) + off
        plsc.store_scatter(cand_v, [pos], v, mask=m)
        return off + plsc.all_reduce_population_count(m)

    cnt = jnp.max(offv)
    padpos = lax.iota(jnp.int32, L) + cnt
    plsc.store_scatter(cand_v, [padpos],
                       jnp.full((L,), -jnp.inf, jnp.float32))

    @pl.when(cnt < K)
    def _():
        @plsc.parallel_loop(0, NV, unroll=UNROLL)
        def _copy(i):
            cand_v[pl.ds(i * L, L)] = row_v[pl.ds(i * L, L)]

    cnt = jnp.where(cnt < K, N, cnt)
    nv = (cnt + (L - 1)) // L

    @plsc.parallel_loop(0, nv, unroll=4)
    def _tokey(i):
        kv = _ckey(cand_v[pl.ds(i * L, L)])
        cand_v[pl.ds(i * L, L)] = lax.bitcast_convert_type(kv, jnp.float32)

    def bit_body(bi, p):
        bit = jnp.left_shift(jnp.int32(1), 31 - bi)
        cand_t = p | bit
        cs = cand_t ^ SIGN

        def cbody(i, acc):
            kv = lax.bitcast_convert_type(cand_v[pl.ds(i * L, L)], jnp.int32)
            return acc + jnp.where(kv >= cs,
                                   jnp.full((L,), 1, jnp.int32),
                                   jnp.full((L,), 0, jnp.int32))

        accv = lax.fori_loop(0, nv, cbody, jnp.zeros((L,), jnp.int32))
        return jnp.where(jnp.sum(accv) >= K, cand_t, p)

    p_u = lax.fori_loop(0, 32, bit_body, jnp.int32(0))
    vkey = p_u ^ SIGN
    bsplat = jnp.full((L,), vkey, jnp.int32)
    bsplat = jnp.where(bsplat >= 0, bsplat, bsplat ^ LOW31)
    return lax.bitcast_convert_type(bsplat, jnp.float32)


def _sc_body(x_hbm, out_hbm, row0_v, row1_v, cand_v,
             in_sem0, in_sem1, out_sem0, out_sem1):
    wid = lax.axis_index("s") * NC + lax.axis_index("c")
    r0 = wid * ROWS_PER_W
    bufs = [row0_v, row1_v]
    in_sems = [in_sem0, in_sem1]
    out_sems = [out_sem0, out_sem1]

    copies_in = [None] * ROWS_PER_W
    copies_out = [None] * ROWS_PER_W
    copies_in[0] = pltpu.async_copy(x_hbm.at[r0], bufs[0], in_sems[0])
    for j in range(ROWS_PER_W):
        b = j % 2
        row_v = bufs[b]
        copies_in[j].wait()
        if j + 1 < ROWS_PER_W:
            if j >= 1:
                copies_out[j - 1].wait()
            copies_in[j + 1] = pltpu.async_copy(
                x_hbm.at[r0 + j + 1], bufs[1 - b], in_sems[1 - b])

        thr = _row_threshold(row_v, cand_v)

        @plsc.parallel_loop(0, NV, unroll=UNROLL)
        def _mask(i):
            v = row_v[pl.ds(i * L, L)]
            row_v[pl.ds(i * L, L)] = jnp.where(v > thr, v, jnp.float32(0.0))

        copies_out[j] = pltpu.async_copy(row_v, out_hbm.at[r0 + j],
                                         out_sems[b])
    copies_out[ROWS_PER_W - 2].wait()
    copies_out[ROWS_PER_W - 1].wait()


@jax.jit
def _ksparse_sc(x):
    mesh = plsc.VectorSubcoreMesh(core_axis_name="c", subcore_axis_name="s")
    return pl.kernel(
        _sc_body,
        out_type=jax.ShapeDtypeStruct((R, N), jnp.float32),
        mesh=mesh,
        compiler_params=pltpu.CompilerParams(needs_layout_passes=False),
        scratch_types=[
            pltpu.VMEM((N,), jnp.float32),
            pltpu.VMEM((N,), jnp.float32),
            pltpu.VMEM((N + L,), jnp.float32),
            pltpu.SemaphoreType.DMA,
            pltpu.SemaphoreType.DMA,
            pltpu.SemaphoreType.DMA,
            pltpu.SemaphoreType.DMA,
        ],
    )(x)


def kernel(inputs):
    return _ksparse_sc(inputs)


# speedup vs baseline: 1.2922x; 1.0525x over previous
import jax
import jax.numpy as jnp
import numpy as np
from jax import lax
from jax.experimental import pallas as pl
from jax.experimental.pallas import tpu as pltpu
from jax.experimental.pallas import tpu_sc as plsc

R = 128
N = 32768
K = 65
L = 16
NV = N // L
NC = 2
NS = 16
NW = NC * NS
ROWS_PER_W = R // NW
PIVOT = np.float32(2.5)
SIGN = np.int32(-(2**31))
LOW31 = np.int32(0x7FFFFFFF)
UNROLL = 8
CAP = 512
CV = CAP // L
NEG_INF = np.float32("-inf")


def _ckey(v):
    b = lax.bitcast_convert_type(v, jnp.int32)
    return jnp.where(b >= 0, b, b ^ LOW31)


def _descend(count_ge):
    def bit_body(bi, p):
        bit = jnp.left_shift(jnp.int32(1), 31 - bi)
        cand_t = p | bit
        cs = cand_t ^ SIGN
        return jnp.where(count_ge(cs) >= K, cand_t, p)

    p_u = lax.fori_loop(0, 32, bit_body, jnp.int32(0))
    return p_u ^ SIGN


def _row_threshold(row_v, cand_v):
    @plsc.parallel_loop(0, CV, unroll=4)
    def _fill(i):
        cand_v[pl.ds(i * L, L)] = jnp.full((L,), NEG_INF, jnp.float32)

    @plsc.parallel_loop(0, NV, unroll=UNROLL,
                        carry=jnp.zeros((L,), jnp.int32))
    def offv(i, off):
        v = row_v[pl.ds(i * L, L)]
        m = v > PIVOT
        mi = m.astype(jnp.int32)
        pos = (plsc.cumsum(mi) - mi) + off
        pos = jnp.minimum(pos, jnp.full((L,), CAP, jnp.int32))
        plsc.store_scatter(cand_v, [pos], v, mask=m)
        return off + plsc.all_reduce_population_count(m)

    cnt = jnp.max(offv)

    def fast_path(_):
        def count_ge(cs):
            acc = jnp.zeros((L,), jnp.int32)
            one = jnp.full((L,), 1, jnp.int32)
            zero = jnp.zeros((L,), jnp.int32)
            for i in range(CV):
                kv = _ckey(cand_v[pl.ds(i * L, L)])
                acc = acc + jnp.where(kv >= cs, one, zero)
            return jnp.sum(acc)

        return _descend(count_ge)

    def slow_path(_):
        def count_ge(cs):
            def cbody(i, acc):
                kv = _ckey(row_v[pl.ds(i * L, L)])
                return acc + jnp.where(kv >= cs,
                                       jnp.full((L,), 1, jnp.int32),
                                       jnp.full((L,), 0, jnp.int32))

            accv = lax.fori_loop(0, NV, cbody, jnp.zeros((L,), jnp.int32))
            return jnp.sum(accv)

        return _descend(count_ge)

    in_window = jnp.logical_and(cnt >= K, cnt <= CAP)
    vkey = lax.cond(in_window, fast_path, slow_path, jnp.int32(0))

    bsplat = jnp.full((L,), vkey, jnp.int32)
    bsplat = jnp.where(bsplat >= 0, bsplat, bsplat ^ LOW31)
    return lax.bitcast_convert_type(bsplat, jnp.float32)


def _sc_body(x_hbm, out_hbm, row0_v, row1_v, row2_v, cand_v,
             in_sem0, in_sem1, in_sem2, out_sem0, out_sem1, out_sem2):
    wid = lax.axis_index("s") * NC + lax.axis_index("c")
    r0 = wid * ROWS_PER_W
    bufs = [row0_v, row1_v, row2_v]
    in_sems = [in_sem0, in_sem1, in_sem2]
    out_sems = [out_sem0, out_sem1, out_sem2]
    nb = len(bufs)

    copies_in = [None] * ROWS_PER_W
    copies_out = [None] * ROWS_PER_W
    for j in range(min(nb, ROWS_PER_W)):
        copies_in[j] = pltpu.async_copy(x_hbm.at[r0 + j], bufs[j],
                                        in_sems[j])
    for j in range(ROWS_PER_W):
        b = j % nb
        row_v = bufs[b]
        copies_in[j].wait()

        thr = _row_threshold(row_v, cand_v)

        jn = j - 1 + nb
        if j >= 1 and jn < ROWS_PER_W:
            bp = (j - 1) % nb
            copies_out[j - 1].wait()
            copies_in[jn] = pltpu.async_copy(x_hbm.at[r0 + jn], bufs[bp],
                                             in_sems[bp])

        @plsc.parallel_loop(0, NV, unroll=UNROLL)
        def _mask(i):
            v = row_v[pl.ds(i * L, L)]
            row_v[pl.ds(i * L, L)] = jnp.where(v > thr, v, jnp.float32(0.0))

        copies_out[j] = pltpu.async_copy(row_v, out_hbm.at[r0 + j],
                                         out_sems[b])
    for j in range(max(0, ROWS_PER_W - nb), ROWS_PER_W):
        if copies_out[j] is not None:
            copies_out[j].wait()


@jax.jit
def _ksparse_sc(x):
    mesh = plsc.VectorSubcoreMesh(core_axis_name="c", subcore_axis_name="s")
    return pl.kernel(
        _sc_body,
        out_type=jax.ShapeDtypeStruct((R, N), jnp.float32),
        mesh=mesh,
        compiler_params=pltpu.CompilerParams(needs_layout_passes=False),
        scratch_types=[
            pltpu.VMEM((N,), jnp.float32),
            pltpu.VMEM((N,), jnp.float32),
            pltpu.VMEM((N,), jnp.float32),
            pltpu.VMEM((CAP + 2 * L,), jnp.float32),
            pltpu.SemaphoreType.DMA,
            pltpu.SemaphoreType.DMA,
            pltpu.SemaphoreType.DMA,
            pltpu.SemaphoreType.DMA,
            pltpu.SemaphoreType.DMA,
            pltpu.SemaphoreType.DMA,
        ],
    )(x)


def kernel(inputs):
    return _ksparse_sc(inputs)
